# baseline (speedup 1.0000x reference)
import functools

import jax
import jax.numpy as jnp
from jax.experimental import pallas as pl
from jax.experimental.pallas import tpu as pltpu

_NEG = -1e30


def _body(nbf_ref, nbe_ref, rel_ref, arg1_ref, fr_ref, fa1_ref, fa2_ref,
          ents_ref, W0_ref, wp0_ref, W1_ref, wp1_ref, out_ref,
          acc_ref, fab_ref, entb_ref, cxm_ref, TR, TF):
    b = pl.program_id(0)
    N = ents_ref.shape[1]
    F = fr_ref.shape[1]
    F2 = 2 * F
    E = fr_ref.shape[2]
    inv2e = 1.0 / (2.0 * E)
    inve = 1.0 / E
    dn = (((1,), (1,)), ((), ()))
    nbf = nbf_ref[b]
    nbe = nbe_ref[b]

    def rowdot(x, Y):
        return jax.lax.dot_general(x, Y, dn,
                                   preferred_element_type=jnp.float32)

    relr = rel_ref[pl.ds(b, 1), :]
    src = arg1_ref[pl.ds(b, 1), :]
    fr = fr_ref[0]
    fa1 = fa1_ref[0]
    fa2 = fa2_ref[0]
    onesf = jnp.ones((1, E), jnp.float32)

    fab_ref[0:F, :] = fa2.astype(jnp.bfloat16)
    fab_ref[F:, :] = fa1.astype(jnp.bfloat16)
    fab = fab_ref[...]

    hops = jnp.dot(relr, jnp.concatenate([W0_ref[...], W1_ref[...]], axis=1),
                   preferred_element_type=jnp.float32)
    hop0 = hops[:, :E]
    hop1 = hops[:, E:]
    ghr = rowdot(jnp.concatenate([hop0, hop1], axis=0), fr)
    fr2 = rowdot(onesf, fr * fr)
    y2cat = rowdot(onesf.astype(jnp.bfloat16), fab * fab)
    gscat = rowdot(src.astype(jnp.bfloat16), fab)
    s2 = jnp.sum(src * src)
    d2rel0 = jnp.sum(hop0 * hop0) + fr2 - 2.0 * ghr[0:1, :]
    d2rel1 = jnp.sum(hop1 * hop1) + fr2 - 2.0 * ghr[1:2, :]
    d2src = s2 + y2cat - 2.0 * gscat
    loga0 = -(d2rel0 + d2src[:, F:]) * inv2e
    loga1 = -(d2rel1 + d2src[:, :F]) * inv2e
    li = jax.lax.broadcasted_iota(jnp.int32, (1, F), 1)
    fvalid = li < nbf
    halfyn = y2cat * 0.5
    loga = jnp.concatenate(
        [jnp.where(fvalid, loga0, _NEG), jnp.where(fvalid, loga1, _NEG)],
        axis=1)
    fin = loga - halfyn * inve

    acc_ref[...] = jnp.full((8, F2), _NEG, jnp.float32)
    for i in range(N // TR):

        @pl.when(i * TR < nbe)
        def _(i=i):
            ef = ents_ref[0, i * TR:(i + 1) * TR, :]
            entb_ref[i * TR:(i + 1) * TR, :] = ef.astype(jnp.bfloat16)
            xn = rowdot(ef * ef, onesf)
            riota = i * TR + jax.lax.broadcasted_iota(jnp.int32, (TR, 1), 0)
            cxm_ref[i * TR:(i + 1) * TR, :] = jnp.where(
                riota < nbe, xn * -0.5, _NEG)

        for k in range(F2 // TF):
            colvalid = ((k * TF) % F) < nbf

            @pl.when((i * TR < nbe) & colvalid)
            def _(i=i, k=k):
                s = rowdot(entb_ref[i * TR:(i + 1) * TR, :],
                           fab_ref[k * TF:(k + 1) * TF, :])
                red = jnp.max(
                    (s + cxm_ref[i * TR:(i + 1) * TR, :]).reshape(
                        TR // 8, 8, TF), axis=0)
                acc_ref[:, k * TF:(k + 1) * TF] = jnp.maximum(
                    acc_ref[:, k * TF:(k + 1) * TF], red)

    a1 = jnp.max(acc_ref[...], axis=0, keepdims=True)
    m_vec = fin + jnp.minimum(a1, halfyn) * inve
    m0 = jnp.max(m_vec[:, :F])
    m1 = jnp.max(m_vec[:, F:])
    p0 = jax.nn.sigmoid(jnp.sum(relr * wp0_ref[...]))
    p1 = jax.nn.sigmoid(jnp.sum(relr * wp1_ref[...]))
    out_ref[0, 0, 0] = jnp.maximum(jnp.minimum(p0, jnp.exp(m0)),
                                   jnp.minimum(p1, jnp.exp(m1)))


def kernel(rel, arg1, arg2, fact_rel, fact_arg1, fact_arg2, nb_facts,
           entity_embeddings, nb_entities, W_hop_0, w_prior_0, W_hop_1,
           w_prior_1):
    B, N, E = entity_embeddings.shape
    F = fact_rel.shape[1]
    TR, TF = 1024, 512
    grid_spec = pltpu.PrefetchScalarGridSpec(
        num_scalar_prefetch=2,
        grid=(B,),
        in_specs=[
            pl.BlockSpec((B, E), lambda b, *_: (0, 0)),
            pl.BlockSpec((B, E), lambda b, *_: (0, 0)),
            pl.BlockSpec((1, F, E), lambda b, *_: (b, 0, 0)),
            pl.BlockSpec((1, F, E), lambda b, *_: (b, 0, 0)),
            pl.BlockSpec((1, F, E), lambda b, *_: (b, 0, 0)),
            pl.BlockSpec((1, N, E), lambda b, *_: (b, 0, 0)),
            pl.BlockSpec((E, E), lambda b, *_: (0, 0)),
            pl.BlockSpec((1, E), lambda b, *_: (0, 0)),
            pl.BlockSpec((E, E), lambda b, *_: (0, 0)),
            pl.BlockSpec((1, E), lambda b, *_: (0, 0)),
        ],
        out_specs=pl.BlockSpec((1, 1, 1), lambda b, *_: (b, 0, 0),
                               memory_space=pltpu.SMEM),
        scratch_shapes=[
            pltpu.VMEM((8, 2 * F), jnp.float32),
            pltpu.VMEM((2 * F, E), jnp.bfloat16),
            pltpu.VMEM((N, E), jnp.bfloat16),
            pltpu.VMEM((N, 1), jnp.float32),
        ],
    )
    out = pl.pallas_call(
        functools.partial(_body, TR=TR, TF=TF),
        grid_spec=grid_spec,
        out_shape=jax.ShapeDtypeStruct((B, 1, 1), jnp.float32),
    )(nb_facts, nb_entities, rel, arg1, fact_rel, fact_arg1, fact_arg2,
      entity_embeddings, W_hop_0, w_prior_0.reshape(1, E), W_hop_1,
      w_prior_1.reshape(1, E))
    return out.reshape(B)
